# baseline (device time: 67875 ns/iter reference)
import jax
import jax.numpy as jnp
from jax import lax
from jax.experimental import pallas as pl
from jax.experimental.pallas import tpu as pltpu

N_DEV = 16
B, SQ, DMODEL = 2, 512, 768
H, DH = 8, 64
HD = H * DH
BLK = 64
LSTRIDE = 8
ROWS = 2 * SQ + (SQ // 32) * LSTRIDE

_MESH = pl.DeviceIdType.MESH


def _allreduce_body(in_ref, out_ref, stag, send_sems, recv_sems):
    my = lax.axis_index("i")

    barrier = pltpu.get_barrier_semaphore()
    for k in range(4):
        pl.semaphore_signal(barrier, inc=1, device_id=(my ^ (1 << k),),
                            device_id_type=_MESH)
    pl.semaphore_wait(barrier, 4)

    out_ref[...] = in_ref[...]

    off = 0
    ln = SQ
    for r in range(4):
        half = ln // 2
        bit = (my >> r) & 1
        send_off = off + (1 - bit) * half
        keep_off = off + bit * half
        partner = my ^ (1 << r)
        lh = (half // 32) * LSTRIDE
        copies = []
        for p, (so, do, n) in enumerate([
                (send_off, 0, half),
                (SQ + send_off, half, half),
                (2 * SQ + (send_off // 32) * LSTRIDE, 2 * half, lh)]):
            copies.append(pltpu.make_async_remote_copy(
                src_ref=out_ref.at[pl.ds(so, n)],
                dst_ref=stag.at[r, pl.ds(do, n)],
                send_sem=send_sems.at[3 * r + p],
                recv_sem=recv_sems.at[3 * r + p],
                device_id=(partner,), device_id_type=_MESH))
        for c in copies:
            c.start()
        for c in copies:
            c.wait()
        for ko, do, n in [(keep_off, 0, half),
                          (SQ + keep_off, half, half),
                          (2 * SQ + (keep_off // 32) * LSTRIDE, 2 * half, lh)]:
            rows = pl.ds(ko, n)
            out_ref[rows] = (
                out_ref[rows].astype(jnp.float32)
                + stag[r, pl.ds(do, n)].astype(jnp.float32)
            ).astype(jnp.bfloat16)
        off = keep_off
        ln = half

    s = off
    seg = ln
    for i, k in enumerate((3, 2, 1, 0)):
        partner = my ^ (1 << k)
        base = 3 * (4 + i)
        lh = (seg // 32) * LSTRIDE
        copies = []
        for p, (so, n) in enumerate([(s, seg), (SQ + s, seg),
                                     (2 * SQ + (s // 32) * LSTRIDE, lh)]):
            copies.append(pltpu.make_async_remote_copy(
                src_ref=out_ref.at[pl.ds(so, n)],
                dst_ref=out_ref.at[pl.ds(so, n)],
                send_sem=send_sems.at[base + p],
                recv_sem=recv_sems.at[base + p],
                device_id=(partner,), device_id_type=_MESH))
        for c in copies:
            c.start()
        for c in copies:
            c.wait()
        s = s - ((my >> k) & 1) * seg
        seg = seg * 2


def _allreduce(packed):
    return pl.pallas_call(
        _allreduce_body,
        out_shape=jax.ShapeDtypeStruct((ROWS, HD), jnp.bfloat16),
        in_specs=[pl.BlockSpec(memory_space=pltpu.VMEM)],
        out_specs=pl.BlockSpec(memory_space=pltpu.VMEM),
        scratch_shapes=[
            pltpu.VMEM((4, SQ + (SQ // 64) * LSTRIDE, HD), jnp.bfloat16),
            pltpu.SemaphoreType.DMA((24,)),
            pltpu.SemaphoreType.DMA((24,)),
        ],
        compiler_params=pltpu.CompilerParams(collective_id=0),
    )(packed)


def kernel(x, Wq, K_ext, V_ext, Wo):
    bf = jnp.bfloat16
    Q = (x.astype(bf) @ Wq.astype(bf)).reshape(B, SQ, H, DH)
    K = K_ext.astype(bf)
    V = V_ext.astype(bf)

    Q4 = Q.reshape(B, 2, 4, BLK, H, DH)
    K4 = K.reshape(B, 2, 4, BLK, H, DH)
    V4 = V.reshape(B, 2, 4, BLK, H, DH)
    s = jnp.einsum("btgihd,bugjhd->bghtiuj", Q4, K4,
                   preferred_element_type=jnp.float32) * 0.125
    w = jnp.exp(s)
    l = jnp.sum(w, axis=(-2, -1))
    num = jnp.einsum("bghtiuj,bugjhd->btgihd", w.astype(bf), V4,
                     preferred_element_type=jnp.float32)
    l = l.transpose(0, 2, 3, 1, 4).reshape(B, H, SQ)

    num2 = num.reshape(B * SQ, HD).astype(bf)
    l_pack = l.transpose(2, 0, 1).reshape(SQ // 32, 1, HD).astype(bf)
    l_pack = jnp.pad(l_pack, ((0, 0), (0, LSTRIDE - 1), (0, 0)))
    packed = jnp.concatenate([num2, l_pack.reshape(-1, HD)], axis=0)

    out = _allreduce(packed)
    num_sum = out[: 2 * SQ].astype(jnp.float32).reshape(B, SQ, H, DH)
    l_sum = out[2 * SQ :].reshape(SQ // 32, LSTRIDE, HD)[:, 0].reshape(SQ, B, H).astype(jnp.float32)

    ctx = num_sum / l_sum.transpose(1, 0, 2)[..., None]
    y = ctx.reshape(B, SQ, HD).astype(bf) @ Wo.astype(bf)
    return y.astype(jnp.float32)


# device time: 67012 ns/iter; 1.0129x vs baseline; 1.0129x over previous
import jax
import jax.numpy as jnp
from jax import lax
from jax.experimental import pallas as pl
from jax.experimental.pallas import tpu as pltpu

N_DEV = 16
B, SQ, DMODEL = 2, 512, 768
H, DH = 8, 64
HD = H * DH
BLK = 64
LSTRIDE = 8
ROWS = 2 * SQ + (SQ // 32) * LSTRIDE

_MESH = pl.DeviceIdType.MESH


def _allreduce_body(in_ref, out_ref, stag, send_sems, recv_sems):
    my = lax.axis_index("i")

    barrier = pltpu.get_barrier_semaphore()
    for xm in (3, 1, 4, 8):
        pl.semaphore_signal(barrier, inc=1, device_id=(my ^ xm,),
                            device_id_type=_MESH)
    pl.semaphore_wait(barrier, 4)

    out_ref[...] = in_ref[...]

    sched = ((3, 1), (1, 0), (4, 2), (8, 3))

    off = 0
    ln = SQ
    for r, (xm, kb) in enumerate(sched):
        half = ln // 2
        bit = (my >> kb) & 1
        send_off = off + (1 - bit) * half
        keep_off = off + bit * half
        partner = my ^ xm
        lh = (half // 32) * LSTRIDE
        copies = []
        for p, (so, do, n) in enumerate([
                (send_off, 0, half),
                (SQ + send_off, half, half),
                (2 * SQ + (send_off // 32) * LSTRIDE, 2 * half, lh)]):
            copies.append(pltpu.make_async_remote_copy(
                src_ref=out_ref.at[pl.ds(so, n)],
                dst_ref=stag.at[r, pl.ds(do, n)],
                send_sem=send_sems.at[3 * r + p],
                recv_sem=recv_sems.at[3 * r + p],
                device_id=(partner,), device_id_type=_MESH))
        for c in copies:
            c.start()
        for c in copies:
            c.wait()
        for ko, do, n in [(keep_off, 0, half),
                          (SQ + keep_off, half, half),
                          (2 * SQ + (keep_off // 32) * LSTRIDE, 2 * half, lh)]:
            rows = pl.ds(ko, n)
            out_ref[rows] = (
                out_ref[rows].astype(jnp.float32)
                + stag[r, pl.ds(do, n)].astype(jnp.float32)
            ).astype(jnp.bfloat16)
        off = keep_off
        ln = half

    s = off
    seg = ln
    for i, (xm, kb) in enumerate(sched[::-1]):
        partner = my ^ xm
        base = 3 * (4 + i)
        lh = (seg // 32) * LSTRIDE
        copies = []
        for p, (so, n) in enumerate([(s, seg), (SQ + s, seg),
                                     (2 * SQ + (s // 32) * LSTRIDE, lh)]):
            copies.append(pltpu.make_async_remote_copy(
                src_ref=out_ref.at[pl.ds(so, n)],
                dst_ref=out_ref.at[pl.ds(so, n)],
                send_sem=send_sems.at[base + p],
                recv_sem=recv_sems.at[base + p],
                device_id=(partner,), device_id_type=_MESH))
        for c in copies:
            c.start()
        for c in copies:
            c.wait()
        s = s - ((my >> kb) & 1) * seg
        seg = seg * 2


def _allreduce(packed):
    return pl.pallas_call(
        _allreduce_body,
        out_shape=jax.ShapeDtypeStruct((ROWS, HD), jnp.bfloat16),
        in_specs=[pl.BlockSpec(memory_space=pltpu.VMEM)],
        out_specs=pl.BlockSpec(memory_space=pltpu.VMEM),
        scratch_shapes=[
            pltpu.VMEM((4, SQ + (SQ // 64) * LSTRIDE, HD), jnp.bfloat16),
            pltpu.SemaphoreType.DMA((24,)),
            pltpu.SemaphoreType.DMA((24,)),
        ],
        compiler_params=pltpu.CompilerParams(collective_id=0),
    )(packed)


def kernel(x, Wq, K_ext, V_ext, Wo):
    bf = jnp.bfloat16
    Q = (x.astype(bf) @ Wq.astype(bf)).reshape(B, SQ, H, DH)
    K = K_ext.astype(bf)
    V = V_ext.astype(bf)

    Q4 = Q.reshape(B, 2, 4, BLK, H, DH)
    K4 = K.reshape(B, 2, 4, BLK, H, DH)
    V4 = V.reshape(B, 2, 4, BLK, H, DH)
    s = jnp.einsum("btgihd,bugjhd->bghtiuj", Q4, K4,
                   preferred_element_type=jnp.float32) * 0.125
    w = jnp.exp(s)
    l = jnp.sum(w, axis=(-2, -1))
    num = jnp.einsum("bghtiuj,bugjhd->btgihd", w.astype(bf), V4,
                     preferred_element_type=jnp.float32)
    l = l.transpose(0, 2, 3, 1, 4).reshape(B, H, SQ)

    num2 = num.reshape(B * SQ, HD).astype(bf)
    l_pack = l.transpose(2, 0, 1).reshape(SQ // 32, 1, HD).astype(bf)
    l_pack = jnp.pad(l_pack, ((0, 0), (0, LSTRIDE - 1), (0, 0)))
    packed = jnp.concatenate([num2, l_pack.reshape(-1, HD)], axis=0)

    out = _allreduce(packed)
    num_sum = out[: 2 * SQ].astype(jnp.float32).reshape(B, SQ, H, DH)
    l_sum = out[2 * SQ :].reshape(SQ // 32, LSTRIDE, HD)[:, 0].reshape(SQ, B, H).astype(jnp.float32)

    ctx = num_sum / l_sum.transpose(1, 0, 2)[..., None]
    y = ctx.reshape(B, SQ, HD).astype(bf) @ Wo.astype(bf)
    return y.astype(jnp.float32)


# device time: 63395 ns/iter; 1.0707x vs baseline; 1.0571x over previous
import jax
import jax.numpy as jnp
from jax import lax
from jax.experimental import pallas as pl
from jax.experimental.pallas import tpu as pltpu

N_DEV = 16
B, SQ, DMODEL = 2, 512, 768
H, DH = 8, 64
HD = H * DH
BLK = 64
LSTRIDE = 8
LROWS = (SQ // 32) * LSTRIDE

_MESH = pl.DeviceIdType.MESH

_SCHED = ((3, 1), (1, 0), (4, 2), (8, 3))


def _allreduce_body(num_ref, l_ref, out_num, out_l,
                    stag, send_sems, recv_sems):
    my = lax.axis_index("i")

    barrier = pltpu.get_barrier_semaphore()
    for xm, _ in _SCHED:
        pl.semaphore_signal(barrier, inc=1, device_id=(my ^ xm,),
                            device_id_type=_MESH)
    pl.semaphore_wait(barrier, 4)

    out_num[...] = num_ref[...]
    out_l[...] = jnp.zeros((LROWS, HD), jnp.bfloat16)
    for g in range(SQ // 32):
        out_l[LSTRIDE * g, :] = l_ref[g, :]

    def round_copies(n_off, half, dst_n0, dst_n1, dst_l, sem_base, partner):
        lh = (half // 32) * LSTRIDE
        out = []
        for p, (src_ref, so, (dst_ref, do), n) in enumerate([
                (out_num, n_off, dst_n0, half),
                (out_num, SQ + n_off, dst_n1, half),
                (out_l, (n_off // 32) * LSTRIDE, dst_l, lh)]):
            out.append(pltpu.make_async_remote_copy(
                src_ref=src_ref.at[pl.ds(so, n)],
                dst_ref=dst_ref.at[pl.ds(do, n)],
                send_sem=send_sems.at[sem_base + p],
                recv_sem=recv_sems.at[sem_base + p],
                device_id=(partner,), device_id_type=_MESH))
        return out

    off = 0
    ln = SQ
    for r, (xm, kb) in enumerate(_SCHED):
        half = ln // 2
        lh = (half // 32) * LSTRIDE
        bit = (my >> kb) & 1
        send_off = off + (1 - bit) * half
        keep_off = off + bit * half
        partner = my ^ xm
        copies = round_copies(
            send_off, half,
            (stag.at[r], 0), (stag.at[r], half), (stag.at[r], 2 * half),
            3 * r, partner)
        for c in copies:
            c.start()
        for c in copies:
            c.wait()
        for ref, ko, do, n in [
                (out_num, keep_off, 0, half),
                (out_num, SQ + keep_off, half, half),
                (out_l, (keep_off // 32) * LSTRIDE, 2 * half, lh)]:
            rows = pl.ds(ko, n)
            ref[rows] = (
                ref[rows].astype(jnp.float32)
                + stag[r, pl.ds(do, n)].astype(jnp.float32)
            ).astype(jnp.bfloat16)
        off = keep_off
        ln = half

    s = off
    seg = ln
    for i, (xm, kb) in enumerate(_SCHED[::-1]):
        partner = my ^ xm
        lh = (seg // 32) * LSTRIDE
        copies = round_copies(
            s, seg,
            (out_num, s), (out_num, SQ + s), (out_l, (s // 32) * LSTRIDE),
            3 * (4 + i), partner)
        for c in copies:
            c.start()
        for c in copies:
            c.wait()
        s = s - ((my >> kb) & 1) * seg
        seg = seg * 2


def _allreduce(num2, l_pack):
    return pl.pallas_call(
        _allreduce_body,
        out_shape=[
            jax.ShapeDtypeStruct((2 * SQ, HD), jnp.bfloat16),
            jax.ShapeDtypeStruct((LROWS, HD), jnp.bfloat16),
        ],
        in_specs=[pl.BlockSpec(memory_space=pltpu.VMEM),
                  pl.BlockSpec(memory_space=pltpu.VMEM)],
        out_specs=[pl.BlockSpec(memory_space=pltpu.VMEM),
                   pl.BlockSpec(memory_space=pltpu.VMEM)],
        scratch_shapes=[
            pltpu.VMEM((4, SQ + (SQ // 64) * LSTRIDE, HD), jnp.bfloat16),
            pltpu.SemaphoreType.DMA((24,)),
            pltpu.SemaphoreType.DMA((24,)),
        ],
        compiler_params=pltpu.CompilerParams(collective_id=0),
    )(num2, l_pack)


def kernel(x, Wq, K_ext, V_ext, Wo):
    bf = jnp.bfloat16
    Q = (x.astype(bf) @ Wq.astype(bf)).reshape(B, SQ, H, DH)
    K = K_ext.astype(bf)
    V = V_ext.astype(bf)

    blk = jnp.arange(SQ) // BLK
    mask = (blk[:, None] % 4) == (blk[None, :] % 4)

    s = jnp.einsum("bihd,bjhd->bhij", Q, K,
                   preferred_element_type=jnp.float32) * 0.125
    w = jnp.where(mask[None, None], jnp.exp(s), 0.0)
    l = jnp.sum(w, axis=-1)
    num = jnp.einsum("bhij,bjhd->bihd", w.astype(bf), V,
                     preferred_element_type=jnp.float32)

    num2 = num.reshape(B * SQ, HD).astype(bf)
    l_pack = l.transpose(2, 0, 1).reshape(SQ // 32, HD).astype(bf)

    num_sum, l_out = _allreduce(num2, l_pack)
    num_sum = num_sum.astype(jnp.float32).reshape(B, SQ, H, DH)
    l_sum = (l_out.reshape(SQ // 32, LSTRIDE, HD)[:, 0]
             .reshape(SQ, B, H).astype(jnp.float32))

    ctx = num_sum / l_sum.transpose(1, 0, 2)[..., None]
    y = ctx.reshape(B, SQ, HD).astype(bf) @ Wo.astype(bf)
    return y.astype(jnp.float32)
